# Initial kernel scaffold; baseline (speedup 1.0000x reference)
import jax
import jax.numpy as jnp
from jax.experimental import pallas as pl


def kernel(det1, det2, homo):
    raise NotImplementedError("write your pallas kernel here")


# speedup vs baseline: 13.1298x; 13.1298x over previous
import functools

import jax
import jax.numpy as jnp
from jax import lax
from jax.experimental import pallas as pl
from jax.experimental.pallas import tpu as pltpu
from jax.experimental.pallas import tpu_sc as plsc

NMS_T = 0.1
TOPK = 512
H = 512
W = 512
NPIX = H * W
NIMG = 8

def _bf16_round(x):
    u = lax.bitcast_convert_type(x, jnp.uint32)
    bias = jnp.uint32(0x7FFF) + ((u >> 16) & jnp.uint32(1))
    u = (u + bias) & jnp.uint32(0xFFFF0000)
    return lax.bitcast_convert_type(u, jnp.float32)


_WORKERS = 32
_PER_IMG = _WORKERS // NIMG
_ROWS_PW = H // _PER_IMG
_CHUNK_ROWS = 2
_CPX = _CHUNK_ROWS * W
_NSLICE = _CPX // 128


def _sc_warp(det2b, xmap, ymap):
    mesh = plsc.VectorSubcoreMesh(core_axis_name="c", subcore_axis_name="s")

    @functools.partial(
        pl.kernel,
        out_type=jax.ShapeDtypeStruct((NIMG, NPIX), jnp.float32),
        mesh=mesh,
        scratch_types=[
            pltpu.VMEM((_CPX,), jnp.float32),
            pltpu.VMEM((_CPX,), jnp.float32),
            pltpu.VMEM((4, _NSLICE, 128), jnp.int32),
            pltpu.VMEM((4, _NSLICE, 128), jnp.int32),
            pltpu.VMEM((4, _NSLICE, 128), jnp.float32),
            pltpu.VMEM((4, _NSLICE, 128, 16), jnp.float32),
            pltpu.VMEM((_CPX,), jnp.float32),
            pltpu.SemaphoreType.DMA,
        ],
        compiler_params=pltpu.CompilerParams(needs_layout_passes=False,
                                             use_tc_tiling_on_sc=False),
    )
    def warp(det2_hbm, x_hbm, y_hbm, out_hbm,
             xb, yb, bidx, laneb, wb, gb, ob, gsem):
        cid = lax.axis_index("c")
        sid = lax.axis_index("s")
        wid = cid * 16 + sid
        img = wid // _PER_IMG
        quad = wid % _PER_IMG
        src = det2_hbm.at[img]
        dst = out_hbm.at[img]
        xsrc = x_hbm.at[img]
        ysrc = y_hbm.at[img]
        row_base = quad * _ROWS_PW
        lanes16 = lax.iota(jnp.int32, 16)

        @pl.loop(0, _ROWS_PW, step=_CHUNK_ROWS)
        def _chunk(rc):
            start = (row_base + rc) * W
            pltpu.sync_copy(xsrc.at[pl.ds(start, _CPX)], xb)
            pltpu.sync_copy(ysrc.at[pl.ds(start, _CPX)], yb)
            for rloc in range(_CHUNK_ROWS):
                @pl.loop(0, W // 16)
                def _vec(kc, rloc=rloc):
                    p = rloc * W + kc * 16
                    xx = xb[pl.ds(p, 16)]
                    yy = yb[pl.ds(p, 16)]
                    xx = jnp.minimum(jnp.maximum(xx, -4.0), 516.0)
                    yy = jnp.minimum(jnp.maximum(yy, -4.0), 516.0)
                    xt = xx.astype(jnp.int32)
                    xtf = xt.astype(jnp.float32)
                    xneg = xx < xtf
                    x0i = xt - xneg.astype(jnp.int32)
                    x0f = xtf - xneg.astype(jnp.float32)
                    yt = yy.astype(jnp.int32)
                    ytf = yt.astype(jnp.float32)
                    yneg = yy < ytf
                    y0i = yt - yneg.astype(jnp.int32)
                    y0f = ytf - yneg.astype(jnp.float32)
                    wx1 = xx - x0f
                    wx0 = 1.0 - wx1
                    wy1 = yy - y0f
                    wy0 = 1.0 - wy1
                    x1f = x0f + 1.0
                    y1f = y0f + 1.0
                    vx0 = jnp.logical_and(x0f >= 0.0, x0f <= 511.0)
                    vx1 = jnp.logical_and(x1f >= 0.0, x1f <= 511.0)
                    vy0 = jnp.logical_and(y0f >= 0.0, y0f <= 511.0)
                    vy1 = jnp.logical_and(y1f >= 0.0, y1f <= 511.0)
                    x0c = jnp.minimum(jnp.maximum(x0i, 0), W - 1)
                    x1c = jnp.minimum(jnp.maximum(x0i + 1, 0), W - 1)
                    y0c = jnp.minimum(jnp.maximum(y0i, 0), H - 1)
                    y1c = jnp.minimum(jnp.maximum(y0i + 1, 0), H - 1)
                    r0 = y0c * W
                    r1 = y1c * W
                    f00 = r0 + x0c
                    f10 = r0 + x1c
                    f01 = r1 + x0c
                    f11 = r1 + x1c
                    zero = jnp.zeros((16,), jnp.float32)
                    sl = rloc * (W // 128) + kc // 8
                    off = (kc % 8) * 16
                    bidx[0, sl, pl.ds(off, 16)] = f00 >> 4
                    bidx[1, sl, pl.ds(off, 16)] = f10 >> 4
                    bidx[2, sl, pl.ds(off, 16)] = f01 >> 4
                    bidx[3, sl, pl.ds(off, 16)] = f11 >> 4
                    laneb[0, sl, pl.ds(off, 16)] = f00 & 15
                    laneb[1, sl, pl.ds(off, 16)] = f10 & 15
                    laneb[2, sl, pl.ds(off, 16)] = f01 & 15
                    laneb[3, sl, pl.ds(off, 16)] = f11 & 15
                    wb[0, sl, pl.ds(off, 16)] = jnp.where(
                        jnp.logical_and(vx0, vy0), wx0 * wy0, zero)
                    wb[1, sl, pl.ds(off, 16)] = jnp.where(
                        jnp.logical_and(vx1, vy0), wx1 * wy0, zero)
                    wb[2, sl, pl.ds(off, 16)] = jnp.where(
                        jnp.logical_and(vx0, vy1), wx0 * wy1, zero)
                    wb[3, sl, pl.ds(off, 16)] = jnp.where(
                        jnp.logical_and(vx1, vy1), wx1 * wy1, zero)

            @pl.loop(0, _NSLICE)
            def _fire(j):
                for t in range(4):
                    pltpu.async_copy(src.at[bidx.at[t, j]], gb.at[t, j], gsem)

            @pl.loop(0, _NSLICE)
            def _drain(j):
                for t in range(4):
                    pltpu.make_async_copy(
                        src.at[pl.ds(0, 128)], gb.at[t, j], gsem).wait()

            @pl.loop(0, _NSLICE)
            def _combine(j):
                @pl.loop(0, 128, step=16)
                def _cv(v):
                    rows = lanes16 + v
                    l0 = laneb[0, j, pl.ds(v, 16)]
                    l1 = laneb[1, j, pl.ds(v, 16)]
                    l2 = laneb[2, j, pl.ds(v, 16)]
                    l3 = laneb[3, j, pl.ds(v, 16)]
                    g0 = plsc.load_gather(gb.at[0, j], [rows, l0])
                    g1 = plsc.load_gather(gb.at[1, j], [rows, l1])
                    g2 = plsc.load_gather(gb.at[2, j], [rows, l2])
                    g3 = plsc.load_gather(gb.at[3, j], [rows, l3])
                    w0 = wb[0, j, pl.ds(v, 16)]
                    w1 = wb[1, j, pl.ds(v, 16)]
                    w2 = wb[2, j, pl.ds(v, 16)]
                    w3 = wb[3, j, pl.ds(v, 16)]
                    ob[pl.ds(j * 128 + v, 16)] = (
                        ((g0 * w0 + g1 * w1) + g2 * w2) + g3 * w3)

            pltpu.sync_copy(ob, dst.at[pl.ds(start, _CPX)])

    return warp(det2b, xmap, ymap)


def _coords_body(hm_ref, x_ref, y_ref):
    h00 = hm_ref[0, 0, 0]; h01 = hm_ref[0, 0, 1]; h02 = hm_ref[0, 0, 2]
    h10 = hm_ref[0, 0, 3]; h11 = hm_ref[0, 0, 4]; h12 = hm_ref[0, 0, 5]
    h20 = hm_ref[0, 0, 6]; h21 = hm_ref[0, 0, 7]; h22 = hm_ref[0, 0, 8]
    rows = lax.broadcasted_iota(jnp.int32, (H, W), 0).astype(jnp.float32)
    cols = lax.broadcasted_iota(jnp.int32, (H, W), 1).astype(jnp.float32)
    rows = _bf16_round(rows)
    cols = _bf16_round(cols)
    zn = ((h20 * cols + h21 * rows) + h22) + 1e-8
    x_ref[0] = ((h00 * cols + h01 * rows) + h02) / zn
    y_ref[0] = ((h10 * cols + h11 * rows) + h12) / zn


def _coords(hm):
    return pl.pallas_call(
        _coords_body,
        grid=(NIMG,),
        in_specs=[pl.BlockSpec((1, 1, 9), lambda i: (i, 0, 0),
                               memory_space=pltpu.SMEM)],
        out_specs=[pl.BlockSpec((1, H, W), lambda i: (i, 0, 0)),
                   pl.BlockSpec((1, H, W), lambda i: (i, 0, 0))],
        out_shape=[jax.ShapeDtypeStruct((NIMG, H, W), jnp.float32),
                   jax.ShapeDtypeStruct((NIMG, H, W), jnp.float32)],
    )(hm.reshape(NIMG, 1, 9))


def _nms_body(det_ref, nms_ref, mask_ref):
    i = pl.program_id(0)
    x = det_ref[0]
    ninf_c = jnp.full((H, 2), -jnp.inf, jnp.float32)
    xp = jnp.concatenate([ninf_c, x, ninf_c], axis=1)
    m = xp[:, 0:W]
    for k in range(1, 5):
        m = jnp.maximum(m, xp[:, k:k + W])
    ninf_r = jnp.full((2, W), -jnp.inf, jnp.float32)
    mp = jnp.concatenate([ninf_r, m, ninf_r], axis=0)
    p = mp[0:H]
    for k in range(1, 5):
        p = jnp.maximum(p, mp[k:k + H])
    keep = jnp.logical_and(x == p, x > NMS_T)
    d = x * keep.astype(jnp.float32)
    nms_ref[0] = d
    di = lax.bitcast_convert_type(d + 0.0, jnp.int32)

    def cnt_gt(b):
        return jnp.sum((di > b).astype(jnp.int32))

    def step(_, lohi):
        lo, hi = lohi
        mid = lo + (hi - lo) // 2
        below = cnt_gt(mid) < TOPK
        return (jnp.where(below, lo, mid + 1), jnp.where(below, mid, hi))

    _, tb = lax.fori_loop(0, 31, step,
                          (jnp.int32(0), jnp.int32(0x7F800000)))
    k0 = (TOPK - cnt_gt(tb)).astype(jnp.float32)
    tie = di == tb
    tief = tie.astype(jnp.float32)
    i0 = lax.broadcasted_iota(jnp.int32, (W, W), 0)
    i1 = lax.broadcasted_iota(jnp.int32, (W, W), 1)
    ut = (i0 < i1).astype(jnp.float32)
    lt = (i0 > i1).astype(jnp.float32)
    pref = jnp.dot(tief, ut, preferred_element_type=jnp.float32,
                   precision=lax.Precision.HIGHEST)
    rowtot = pref[:, W - 1:W] + tief[:, W - 1:W]
    rowpre = jnp.dot(lt, rowtot, preferred_element_type=jnp.float32,
                     precision=lax.Precision.HIGHEST)
    rank = rowpre + pref
    sel = jnp.logical_or(di > tb, jnp.logical_and(tie, rank < k0))
    selm = sel.astype(jnp.float32)

    @pl.when(i == 0)
    def _():
        mask_ref[...] = selm

    @pl.when(i != 0)
    def _():
        mask_ref[...] = jnp.maximum(mask_ref[...], selm)


def _nms_topk(det):
    return pl.pallas_call(
        _nms_body,
        grid=(NIMG,),
        in_specs=[pl.BlockSpec((1, H, W), lambda i: (i, 0, 0))],
        out_specs=[pl.BlockSpec((1, H, W), lambda i: (i, 0, 0)),
                   pl.BlockSpec((H, W), lambda i: (0, 0))],
        out_shape=[jax.ShapeDtypeStruct((NIMG, H, W), jnp.float32),
                   jax.ShapeDtypeStruct((H, W), jnp.float32)],
    )(det)


def _gauss_taps():
    import numpy as np
    ax = np.arange(5, dtype=np.float32) - 2.0
    g = np.exp(-(ax ** 2) / (2.0 * 1.5 ** 2)).astype(np.float32)
    g = (g / g.sum()).astype(np.float32)
    return [float(v) for v in g]


_G = _gauss_taps()


def _gauss2d(d):
    zc = jnp.zeros((H, 2), jnp.float32)
    dp = jnp.concatenate([zc, d, zc], axis=1)
    c = dp[:, 0:W] * _G[0]
    for k in range(1, 5):
        c = c + dp[:, k:k + W] * _G[k]
    zr = jnp.zeros((2, W), jnp.float32)
    cp = jnp.concatenate([zr, c, zr], axis=0)
    o = cp[0:H] * _G[0]
    for k in range(1, 5):
        o = o + cp[k:k + H] * _G[k]
    return o


def _final_body(hm_ref, nms1_ref, nms2_ref, m1_ref, m2_ref,
                vis_ref, topk_ref, num_ref, den_ref):
    i = pl.program_id(0)
    h00 = hm_ref[0, 0, 0]; h01 = hm_ref[0, 0, 1]; h02 = hm_ref[0, 0, 2]
    h10 = hm_ref[0, 0, 3]; h11 = hm_ref[0, 0, 4]; h12 = hm_ref[0, 0, 5]
    h20 = hm_ref[0, 0, 6]; h21 = hm_ref[0, 0, 7]; h22 = hm_ref[0, 0, 8]
    rows = lax.broadcasted_iota(jnp.int32, (H, W), 0).astype(jnp.float32)
    cols = lax.broadcasted_iota(jnp.int32, (H, W), 1).astype(jnp.float32)
    rows = _bf16_round(rows)
    cols = _bf16_round(cols)
    xn = (h00 * cols + h01 * rows) + h02
    yn = (h10 * cols + h11 * rows) + h12
    zn = ((h20 * cols + h21 * rows) + h22) + 1e-8
    x = xn / zn
    y = yn / zn
    x0 = jnp.floor(x)
    y0 = jnp.floor(y)
    x1 = x0 + 1.0
    y1 = y0 + 1.0
    wx1 = x - x0
    wx0 = 1.0 - wx1
    wy1 = y - y0
    wy0 = 1.0 - wy1
    vx0 = jnp.logical_and(x0 >= 0.0, x0 <= W - 1.0)
    vx1 = jnp.logical_and(x1 >= 0.0, x1 <= W - 1.0)
    vy0 = jnp.logical_and(y0 >= 0.0, y0 <= H - 1.0)
    vy1 = jnp.logical_and(y1 >= 0.0, y1 <= H - 1.0)
    f = jnp.float32
    v00 = jnp.logical_and(vx0, vy0).astype(f)
    v10 = jnp.logical_and(vx1, vy0).astype(f)
    v01 = jnp.logical_and(vx0, vy1).astype(f)
    v11 = jnp.logical_and(vx1, vy1).astype(f)
    vis = (v00 * (wx0 * wy0) + v10 * (wx1 * wy0)
           + v01 * (wx0 * wy1) + v11 * (wx1 * wy1))
    vb = (vis > 0.0).astype(f)
    pinf_c = jnp.full((H, 1), jnp.inf, jnp.float32)
    vp = jnp.concatenate([pinf_c, vb, pinf_c], axis=1)
    e = jnp.minimum(jnp.minimum(vp[:, 0:W], vp[:, 1:W + 1]), vp[:, 2:W + 2])
    pinf_r = jnp.full((1, W), jnp.inf, jnp.float32)
    ep = jnp.concatenate([pinf_r, e, pinf_r], axis=0)
    er = jnp.minimum(jnp.minimum(ep[0:H], ep[1:H + 1]), ep[2:H + 2])
    vis_ref[0] = er
    m2 = m2_ref[...]
    topk_ref[0] = m2
    g1 = _gauss2d(nms1_ref[0] * m1_ref[...])
    g2 = _gauss2d(nms2_ref[0] * m2)
    df = g1 - g2
    pn = jnp.sum(df * df * er)
    pd = jnp.sum(er)

    @pl.when(i == 0)
    def _():
        num_ref[0, 0] = pn
        den_ref[0, 0] = pd

    @pl.when(i != 0)
    def _():
        num_ref[0, 0] = num_ref[0, 0] + pn
        den_ref[0, 0] = den_ref[0, 0] + pd


def _final(hm, nms1, nms2, mask1, mask2):
    return pl.pallas_call(
        _final_body,
        grid=(NIMG,),
        in_specs=[
            pl.BlockSpec((1, 1, 9), lambda i: (i, 0, 0),
                         memory_space=pltpu.SMEM),
            pl.BlockSpec((1, H, W), lambda i: (i, 0, 0)),
            pl.BlockSpec((1, H, W), lambda i: (i, 0, 0)),
            pl.BlockSpec((H, W), lambda i: (0, 0)),
            pl.BlockSpec((H, W), lambda i: (0, 0)),
        ],
        out_specs=[pl.BlockSpec((1, H, W), lambda i: (i, 0, 0)),
                   pl.BlockSpec((1, H, W), lambda i: (i, 0, 0)),
                   pl.BlockSpec((1, 1), lambda i: (0, 0),
                                memory_space=pltpu.SMEM),
                   pl.BlockSpec((1, 1), lambda i: (0, 0),
                                memory_space=pltpu.SMEM)],
        out_shape=[jax.ShapeDtypeStruct((NIMG, H, W), jnp.float32),
                   jax.ShapeDtypeStruct((NIMG, H, W), jnp.float32),
                   jax.ShapeDtypeStruct((1, 1), jnp.float32),
                   jax.ShapeDtypeStruct((1, 1), jnp.float32)],
    )(hm.reshape(NIMG, 1, 9), nms1, nms2, mask1, mask2)


def kernel(det1, det2, homo):
    det1r = det1.reshape(NIMG, H, W).astype(jnp.float32)
    det2b = det2.reshape(NIMG, NPIX // 16, 16).astype(jnp.float32)
    hm = _bf16_round(homo.reshape(NIMG, 9).astype(jnp.float32))
    xmap, ymap = _coords(hm)
    wd2 = _sc_warp(det2b, xmap.reshape(NIMG, NPIX), ymap.reshape(NIMG, NPIX))
    nms1, mask1 = _nms_topk(det1r)
    nms2, mask2 = _nms_topk(wd2.reshape(NIMG, H, W))
    vis, topk, num, den = _final(hm, nms1, nms2, mask1, mask2)
    loss = num[0, 0] / den[0, 0]
    return (loss,
            topk.reshape(NIMG, 1, H, W),
            vis.reshape(NIMG, 1, H, W))
